# baseline (speedup 1.0000x reference)
import jax
import jax.numpy as jnp
from jax.experimental import pallas as pl

_B = 16384
_D = 128
_R = 1024


def _body(ids_ref, g_ref, w_ref, t_ref, b_ref, o_ref):
    ids = ids_ref[0]
    cols = jax.lax.broadcasted_iota(jnp.int32, (_R, 32), 1)
    oh = (
        (cols == ids[0][:, None])
        | (cols == ids[1][:, None])
        | (cols == ids[2][:, None])
    ).astype(jnp.float32)
    acc = jnp.dot(g_ref[...], w_ref[...], preferred_element_type=jnp.float32)
    acc += jnp.dot(oh, t_ref[...], preferred_element_type=jnp.float32)
    o_ref[...] = acc + b_ref[...]


def kernel(style_ids, key_ids, section_ids, groove_features, style_table,
           key_table, section_table, groove_W, groove_b):
    nb = _B // _R
    ids3 = jnp.stack(
        [style_ids.astype(jnp.int32),
         key_ids.astype(jnp.int32) + 3,
         section_ids.astype(jnp.int32) + 27],
        axis=0,
    )
    ids3 = ids3.reshape(3, nb, _R).transpose(1, 0, 2)
    tables = jnp.concatenate([style_table, key_table, section_table], axis=0)
    bias = groove_b.reshape(1, _D)

    return pl.pallas_call(
        _body,
        grid=(nb,),
        in_specs=[
            pl.BlockSpec((1, 3, _R), lambda i: (i, 0, 0)),
            pl.BlockSpec((_R, 32), lambda i: (i, 0)),
            pl.BlockSpec((32, _D), lambda i: (0, 0)),
            pl.BlockSpec((32, _D), lambda i: (0, 0)),
            pl.BlockSpec((1, _D), lambda i: (0, 0)),
        ],
        out_specs=pl.BlockSpec((_R, _D), lambda i: (i, 0)),
        out_shape=jax.ShapeDtypeStruct((_B, _D), jnp.float32),
    )(ids3, groove_features, groove_W, tables, bias)


# speedup vs baseline: 1.2524x; 1.2524x over previous
import jax
import jax.numpy as jnp
from jax.experimental import pallas as pl

_B = 16384
_D = 128
_R = 8192


def _body(ids_ref, g_ref, w_ref, t_ref, b_ref, o_ref):
    ids = ids_ref[0]
    cols = jax.lax.broadcasted_iota(jnp.int32, (_R, 32), 1)
    oh = (
        (cols == ids[0][:, None])
        | (cols == ids[1][:, None])
        | (cols == ids[2][:, None])
    ).astype(jnp.float32)
    acc = jnp.dot(g_ref[...], w_ref[...], preferred_element_type=jnp.float32)
    acc += jnp.dot(oh, t_ref[...], preferred_element_type=jnp.float32)
    o_ref[...] = acc + b_ref[...]


def kernel(style_ids, key_ids, section_ids, groove_features, style_table,
           key_table, section_table, groove_W, groove_b):
    nb = _B // _R
    ids3 = jnp.stack(
        [style_ids.astype(jnp.int32),
         key_ids.astype(jnp.int32) + 3,
         section_ids.astype(jnp.int32) + 27],
        axis=0,
    )
    ids3 = ids3.reshape(3, nb, _R).transpose(1, 0, 2)
    tables = jnp.concatenate([style_table, key_table, section_table], axis=0)
    bias = groove_b.reshape(1, _D)

    return pl.pallas_call(
        _body,
        grid=(nb,),
        in_specs=[
            pl.BlockSpec((1, 3, _R), lambda i: (i, 0, 0)),
            pl.BlockSpec((_R, 32), lambda i: (i, 0)),
            pl.BlockSpec((32, _D), lambda i: (0, 0)),
            pl.BlockSpec((32, _D), lambda i: (0, 0)),
            pl.BlockSpec((1, _D), lambda i: (0, 0)),
        ],
        out_specs=pl.BlockSpec((_R, _D), lambda i: (i, 0)),
        out_shape=jax.ShapeDtypeStruct((_B, _D), jnp.float32),
    )(ids3, groove_features, groove_W, tables, bias)


# speedup vs baseline: 1.5427x; 1.2317x over previous
import jax
import jax.numpy as jnp
from jax.experimental import pallas as pl

_B = 16384
_D = 128
_R = 8192


def _body(ids_ref, g_ref, w_ref, t_ref, b_ref, o_ref):
    ids = ids_ref[...]
    cols = jax.lax.broadcasted_iota(jnp.int32, (32, _R), 0)
    ohT = (
        (cols == ids[0:1]) | (cols == ids[1:2]) | (cols == ids[2:3])
    ).astype(jnp.float32)
    acc = jax.lax.dot_general(
        ohT, t_ref[...], (((0,), (0,)), ((), ())),
        preferred_element_type=jnp.float32,
    )
    acc += jnp.dot(g_ref[...], w_ref[...], preferred_element_type=jnp.float32)
    o_ref[...] = acc + b_ref[...]


def kernel(style_ids, key_ids, section_ids, groove_features, style_table,
           key_table, section_table, groove_W, groove_b):
    nb = _B // _R
    ids3 = jnp.stack(
        [style_ids.astype(jnp.int32),
         key_ids.astype(jnp.int32) + 3,
         section_ids.astype(jnp.int32) + 27],
        axis=0,
    )
    tables = jnp.concatenate([style_table, key_table, section_table], axis=0)
    bias = groove_b.reshape(1, _D)

    return pl.pallas_call(
        _body,
        grid=(nb,),
        in_specs=[
            pl.BlockSpec((3, _R), lambda i: (0, i)),
            pl.BlockSpec((_R, 32), lambda i: (i, 0)),
            pl.BlockSpec((32, _D), lambda i: (0, 0)),
            pl.BlockSpec((32, _D), lambda i: (0, 0)),
            pl.BlockSpec((1, _D), lambda i: (0, 0)),
        ],
        out_specs=pl.BlockSpec((_R, _D), lambda i: (i, 0)),
        out_shape=jax.ShapeDtypeStruct((_B, _D), jnp.float32),
    )(ids3, groove_features, groove_W, tables, bias)


# speedup vs baseline: 1.8963x; 1.2292x over previous
import jax
import jax.numpy as jnp
from jax.experimental import pallas as pl

_B = 16384
_D = 128
_R = 8192


def _body(s_ref, k_ref, c_ref, g_ref, w_ref, t_ref, b_ref, o_ref):
    cols = jax.lax.broadcasted_iota(jnp.int32, (32, _R), 0)
    ohT = (
        (cols == s_ref[0])
        | ((cols - 3) == k_ref[0])
        | ((cols - 27) == c_ref[0])
    ).astype(jnp.float32)
    acc = jax.lax.dot_general(
        ohT, t_ref[...], (((0,), (0,)), ((), ())),
        preferred_element_type=jnp.float32,
    )
    acc += jnp.dot(g_ref[...], w_ref[...], preferred_element_type=jnp.float32)
    o_ref[...] = acc + b_ref[...]


def kernel(style_ids, key_ids, section_ids, groove_features, style_table,
           key_table, section_table, groove_W, groove_b):
    nb = _B // _R
    sid = style_ids.astype(jnp.int32).reshape(nb, 1, _R)
    kid = key_ids.astype(jnp.int32).reshape(nb, 1, _R)
    cid = section_ids.astype(jnp.int32).reshape(nb, 1, _R)
    tables = jnp.concatenate([style_table, key_table, section_table], axis=0)
    bias = groove_b.reshape(1, _D)

    idspec = pl.BlockSpec((1, 1, _R), lambda i: (i, 0, 0))
    return pl.pallas_call(
        _body,
        grid=(nb,),
        in_specs=[
            idspec, idspec, idspec,
            pl.BlockSpec((_R, 32), lambda i: (i, 0)),
            pl.BlockSpec((32, _D), lambda i: (0, 0)),
            pl.BlockSpec((32, _D), lambda i: (0, 0)),
            pl.BlockSpec((1, _D), lambda i: (0, 0)),
        ],
        out_specs=pl.BlockSpec((_R, _D), lambda i: (i, 0)),
        out_shape=jax.ShapeDtypeStruct((_B, _D), jnp.float32),
    )(sid, kid, cid, groove_features, groove_W, tables, bias)


# speedup vs baseline: 2.0831x; 1.0985x over previous
import jax
import jax.numpy as jnp
from jax.experimental import pallas as pl

_B = 16384
_D = 128
_R = 8192


def _body(s_ref, k_ref, c_ref, g_ref, w_ref, t1_ref, t2_ref, t3_ref, b_ref, o_ref):
    cols = jax.lax.broadcasted_iota(jnp.int32, (32, _R), 0)
    ohT = (
        (cols == s_ref[0])
        | ((cols - 3) == k_ref[0])
        | ((cols - 27) == c_ref[0])
    ).astype(jnp.float32)
    tables = jnp.concatenate([t1_ref[...], t2_ref[...], t3_ref[...]], axis=0)
    acc = jax.lax.dot_general(
        ohT, tables, (((0,), (0,)), ((), ())),
        preferred_element_type=jnp.float32,
    )
    acc += jnp.dot(g_ref[...], w_ref[...], preferred_element_type=jnp.float32)
    o_ref[...] = acc + b_ref[...]


def kernel(style_ids, key_ids, section_ids, groove_features, style_table,
           key_table, section_table, groove_W, groove_b):
    nb = _B // _R
    sid = style_ids.astype(jnp.int32).reshape(nb, 1, _R)
    kid = key_ids.astype(jnp.int32).reshape(nb, 1, _R)
    cid = section_ids.astype(jnp.int32).reshape(nb, 1, _R)
    bias = groove_b.reshape(1, _D)

    idspec = pl.BlockSpec((1, 1, _R), lambda i: (i, 0, 0))
    return pl.pallas_call(
        _body,
        grid=(nb,),
        in_specs=[
            idspec, idspec, idspec,
            pl.BlockSpec((_R, 32), lambda i: (i, 0)),
            pl.BlockSpec((32, _D), lambda i: (0, 0)),
            pl.BlockSpec((3, _D), lambda i: (0, 0)),
            pl.BlockSpec((24, _D), lambda i: (0, 0)),
            pl.BlockSpec((5, _D), lambda i: (0, 0)),
            pl.BlockSpec((1, _D), lambda i: (0, 0)),
        ],
        out_specs=pl.BlockSpec((_R, _D), lambda i: (i, 0)),
        out_shape=jax.ShapeDtypeStruct((_B, _D), jnp.float32),
    )(sid, kid, cid, groove_features, groove_W,
      style_table, key_table, section_table, bias)
